# baseline (speedup 1.0000x reference)
import jax
import jax.numpy as jnp
from jax.experimental import pallas as pl
from jax.experimental.pallas import tpu as pltpu

_RB = 2048
_SUB = 512
_NBUF = 2
_NOUT = 4
_NE = 8


def _top2_block(x, wt):
    logits = jnp.dot(x, wt, preferred_element_type=jnp.float32)
    lane = jax.lax.broadcasted_iota(jnp.int32, logits.shape, 1)
    l1 = jnp.max(logits, axis=-1, keepdims=True)
    i1 = jnp.argmax(logits, axis=-1).astype(jnp.int32)[:, None]
    masked = jnp.where(lane == i1, -jnp.inf, logits)
    l2 = jnp.max(masked, axis=-1, keepdims=True)
    i2 = jnp.argmax(masked, axis=-1).astype(jnp.int32)[:, None]
    t = jnp.exp(l2 - l1)
    w1 = 1.0 / (1.0 + t)
    w2 = t * w1
    idx = jnp.concatenate([i1, i2], axis=1)
    w = jnp.concatenate([w1, w2], axis=1)
    return idx, w


def _gate_body(x_hbm, wt_ref, idx_hbm, w_hbm, xbuf, ibuf, wbuf, sems, isems, wsems):
    nd = x_hbm.shape[0]
    rows = x_hbm.shape[1]
    nch = rows // _RB

    def copy(j, slot):
        return pltpu.make_async_copy(
            x_hbm.at[:, pl.ds(j * _RB, _RB), :], xbuf.at[slot], sems.at[slot]
        )

    def out_copy(t0, oslot):
        return (
            pltpu.make_async_copy(
                ibuf.at[oslot], idx_hbm.at[pl.ds(t0, _SUB), :], isems.at[oslot]
            ),
            pltpu.make_async_copy(
                wbuf.at[oslot], w_hbm.at[pl.ds(t0, _SUB), :], wsems.at[oslot]
            ),
        )

    for s in range(_NBUF):
        copy(s, s).start()

    nsub = _RB // _SUB
    step = 0

    def loop(j, carry):
        slot = jax.lax.rem(j, _NBUF)
        copy(j, slot).wait()
        for d in range(nd):
            def sub(sb, c):
                g = (j * nd + d) * nsub + sb
                t0 = d * rows + j * _RB + sb * _SUB
                oslot = jax.lax.rem(g, _NOUT)
                idx, w = _top2_block(
                    xbuf[slot, d, pl.ds(sb * _SUB, _SUB), :], wt_ref[...]
                )

                @pl.when(g >= _NOUT)
                def _():
                    gp = g - _NOUT
                    sbp = jax.lax.rem(gp, nsub)
                    dn = jax.lax.rem(gp // nsub, nd)
                    jp = gp // (nsub * nd)
                    t0p = dn * rows + jp * _RB + sbp * _SUB
                    pic, pwc = out_copy(t0p, oslot)
                    pic.wait()
                    pwc.wait()

                ibuf[oslot] = idx
                wbuf[oslot] = w
                ic, wc = out_copy(t0, oslot)
                ic.start()
                wc.start()
                return c

            jax.lax.fori_loop(0, nsub, sub, 0)

        @pl.when(j + _NBUF < nch)
        def _():
            copy(j + _NBUF, slot).start()

        return carry

    jax.lax.fori_loop(0, nch, loop, 0)

    total = nch * nd * nsub
    for k in range(_NOUT):
        g = total - _NOUT + k
        oslot = g % _NOUT
        sbp = g % nsub
        dn = (g // nsub) % nd
        jp = g // (nsub * nd)
        t0p = dn * rows + jp * _RB + sbp * _SUB
        pic, pwc = out_copy(t0p, oslot)
        pic.wait()
        pwc.wait()


def _route(x3, wt):
    nd, rows, h = x3.shape
    n = nd * rows
    return pl.pallas_call(
        _gate_body,
        in_specs=[
            pl.BlockSpec(memory_space=pl.ANY),
            pl.BlockSpec(memory_space=pltpu.VMEM),
        ],
        out_specs=[
            pl.BlockSpec(memory_space=pl.ANY),
            pl.BlockSpec(memory_space=pl.ANY),
        ],
        out_shape=[
            jax.ShapeDtypeStruct((n, 2), jnp.int32),
            jax.ShapeDtypeStruct((n, 2), jnp.float32),
        ],
        scratch_shapes=[
            pltpu.VMEM((_NBUF, nd, _RB, h), jnp.float32),
            pltpu.VMEM((_NOUT, _SUB, 2), jnp.int32),
            pltpu.VMEM((_NOUT, _SUB, 2), jnp.float32),
            pltpu.SemaphoreType.DMA((_NBUF,)),
            pltpu.SemaphoreType.DMA((_NOUT,)),
            pltpu.SemaphoreType.DMA((_NOUT,)),
        ],
        compiler_params=pltpu.CompilerParams(
            vmem_limit_bytes=62 * 1024 * 1024,
        ),
    )(x3, wt)


@jax.jit
def kernel(hidden_states, weight):
    topk_idx, topk_weight = _route(hidden_states, weight.T)
    return topk_idx, topk_weight


# speedup vs baseline: 1.1383x; 1.1383x over previous
import jax
import jax.numpy as jnp
from jax.experimental import pallas as pl
from jax.experimental.pallas import tpu as pltpu

_BLOCK = 4096
_NE = 8


def _gate_body(x_ref, wt_ref, idx_ref, w_ref):
    x = x_ref[...]
    logits = jnp.dot(x, wt_ref[...], preferred_element_type=jnp.float32)
    lane = jax.lax.broadcasted_iota(jnp.int32, logits.shape, 1)
    l1 = jnp.max(logits, axis=-1, keepdims=True)
    i1 = jnp.argmax(logits, axis=-1).astype(jnp.int32)[:, None]
    masked = jnp.where(lane == i1, -jnp.inf, logits)
    l2 = jnp.max(masked, axis=-1, keepdims=True)
    i2 = jnp.argmax(masked, axis=-1).astype(jnp.int32)[:, None]
    t = jnp.exp(l2 - l1)
    w1 = 1.0 / (1.0 + t)
    w2 = t * w1
    idx_ref[...] = jnp.concatenate([i1, i2], axis=1)
    w_ref[...] = jnp.concatenate([w1, w2], axis=1)


def _route(x, wt):
    n = x.shape[0]
    grid = n // _BLOCK
    return pl.pallas_call(
        _gate_body,
        grid=(grid,),
        in_specs=[
            pl.BlockSpec((_BLOCK, x.shape[1]), lambda i: (i, 0)),
            pl.BlockSpec((x.shape[1], _NE), lambda i: (0, 0)),
        ],
        out_specs=[
            pl.BlockSpec((_BLOCK, 2), lambda i: (i, 0)),
            pl.BlockSpec((_BLOCK, 2), lambda i: (i, 0)),
        ],
        out_shape=[
            jax.ShapeDtypeStruct((n, 2), jnp.int32),
            jax.ShapeDtypeStruct((n, 2), jnp.float32),
        ],
        compiler_params=pltpu.CompilerParams(
            dimension_semantics=("arbitrary",),
        ),
    )(x, wt)


@jax.jit
def kernel(hidden_states, weight):
    h = hidden_states.shape[-1]
    x = hidden_states.reshape(-1, h)
    topk_idx, topk_weight = _route(x, weight.T)
    return topk_idx, topk_weight
